# baseline (speedup 1.0000x reference)
import jax
import jax.numpy as jnp
import numpy as np
from jax.experimental import pallas as pl
from jax.experimental.pallas import tpu as pltpu

BSZ, N, STEPS, FDIM = 2, 512, 96, 16
SDIM, H, E_H, NSP, TOPK = 32, 256, 64, 3, 2
RATIO, MINE, MAXE, NLAYERS = 0.1, 8, 64, 2
KSZ = int(np.clip(int(N * RATIO), MINE, MAXE))
BN = BSZ * N

_SQRT_HALF = 0.7071067811865476


def _gelu(x):
    return 0.5 * x * (1.0 + jax.lax.erf(x * _SQRT_HALF))


UNROLL = 4


def _gru_body(f_ref, s_ref, ws1_ref, bs1_ref, ws2_ref, bs2_ref,
              wih_ref, whh_ref, brz_ref, bin_ref, bhn_ref,
              hs_out, ht_out, h1_scr, h2_scr):
    t = pl.program_id(0)
    HB = BN // 2

    @pl.when(t == 0)
    def _init():
        s = s_ref[...]
        h0 = _gelu(jnp.dot(s, ws1_ref[...], preferred_element_type=jnp.float32)
                   + bs1_ref[...])
        h0 = jnp.dot(h0, ws2_ref[...], preferred_element_type=jnp.float32) + bs2_ref[...]
        hs_out[...] = h0
        h1_scr[...] = h0[:HB]
        h2_scr[...] = h0[HB:]

    wih = wih_ref[...]
    whh = whh_ref[...]
    brz = brz_ref[...]
    bin_ = bin_ref[...]
    bhn = bhn_ref[...]
    hs = [h1_scr[...], h2_scr[...]]
    for j in range(UNROLL):
        for k, lo in enumerate((0, HB)):
            h = hs[k]
            xt = f_ref[j, pl.ds(lo, HB), :].astype(jnp.bfloat16)
            gi = jnp.dot(xt, wih, preferred_element_type=jnp.float32)
            gh = jnp.dot(h.astype(jnp.bfloat16), whh,
                         preferred_element_type=jnp.float32)
            rz = jnp.tanh((gi[:, :2 * H] + gh[:, :2 * H] + brz) * 0.5)
            r = rz[:, :H] * 0.5 + 0.5
            z = rz[:, H:] * 0.5 + 0.5
            n = jnp.tanh(gi[:, 2 * H:] + bin_ + r * (gh[:, 2 * H:] + bhn))
            hs[k] = n + z * (h - n)
    h1_scr[...] = hs[0]
    h2_scr[...] = hs[1]

    @pl.when(t == STEPS // UNROLL - 1)
    def _fin():
        ht_out[pl.ds(0, HB), :] = hs[0]
        ht_out[pl.ds(HB, HB), :] = hs[1]


def _run_gru(f_t, s, p):
    const2 = lambda shape: pl.BlockSpec(shape, lambda t: (0, 0))
    grid = (STEPS // UNROLL,)
    out = pl.pallas_call(
        _gru_body,
        grid=grid,
        in_specs=[
            pl.BlockSpec((UNROLL, BN, FDIM), lambda t: (t, 0, 0)),
            const2((BN, SDIM)),
            const2((SDIM, H)), const2((1, H)),
            const2((H, H)), const2((1, H)),
            pl.BlockSpec((FDIM, 3 * H), lambda t: (0, 0)),
            pl.BlockSpec((H, 3 * H), lambda t: (0, 0)),
            const2((1, 2 * H)), const2((1, H)), const2((1, H)),
        ],
        out_specs=[
            pl.BlockSpec((BN, H), lambda t: (0, 0)),
            pl.BlockSpec((BN, H), lambda t: (0, 0)),
        ],
        out_shape=[
            jax.ShapeDtypeStruct((BN, H), jnp.float32),
            jax.ShapeDtypeStruct((BN, H), jnp.float32),
        ],
        scratch_shapes=[pltpu.VMEM((BN // 2, H), jnp.float32),
                        pltpu.VMEM((BN // 2, H), jnp.float32)],
        compiler_params=pltpu.CompilerParams(
            dimension_semantics=("arbitrary",)),
    )(
        f_t, s,
        p['static1']['w'], p['static1']['b'][None, :],
        p['static2']['w'], p['static2']['b'][None, :],
        p['gru_wih'].astype(jnp.bfloat16), p['gru_whh'].astype(jnp.bfloat16),
        (p['gru_bih'][:2 * H] + p['gru_bhh'][:2 * H])[None, :],
        p['gru_bih'][2 * H:][None, :], p['gru_bhh'][2 * H:][None, :],
    )
    return out


def _float_key(x):
    b = jax.lax.bitcast_convert_type(x, jnp.int32)
    mag = jnp.bitwise_and(b, jnp.int32(0x7FFFFFFF))
    return jnp.where(b < 0, -mag, b)


def _kth_threshold_keys(keys, k):
    cols = keys.shape[1]
    lo = jnp.full((1, cols), -2139095041, jnp.int32)
    hi = jnp.full((1, cols), 2139095041, jnp.int32)

    def body(_, lh):
        lo, hi = lh
        mid = jnp.bitwise_and(lo, hi) + (jnp.bitwise_xor(lo, hi) >> 1)
        cnt = jnp.sum((keys >= mid).astype(jnp.int32), axis=0, keepdims=True)
        take = cnt >= k
        return jnp.where(take, mid, lo), jnp.where(take, hi, mid)

    lo, hi = jax.lax.fori_loop(0, 33, body, (lo, hi))
    return lo


def _generator(view, pw_ref, pb_ref, pe_ref):
    q = _gelu(jnp.dot(view, pw_ref[...], preferred_element_type=jnp.float32)
              + pb_ref[...])
    logits = jnp.dot(q, pe_ref[...], preferred_element_type=jnp.float32)
    keys = _float_key(logits)
    thr = _kth_threshold_keys(keys, KSZ)
    sel = (keys >= thr).astype(jnp.float32)
    m = jnp.max(logits, axis=0, keepdims=True)
    e = jnp.exp(logits - m) * sel
    inc = e / jnp.sum(e, axis=0, keepdims=True)
    cnorm = jnp.sqrt(jnp.sum(inc * inc, axis=0, keepdims=True)) + 1e-8
    cols = inc / cnorm
    sim = jax.lax.dot_general(cols, cols, (((0,), (0,)), ((), ())),
                              preferred_element_type=jnp.float32)
    ri = jax.lax.broadcasted_iota(jnp.int32, (E_H, E_H), 0)
    ci = jax.lax.broadcasted_iota(jnp.int32, (E_H, E_H), 1)
    offd = (ri != ci).astype(jnp.float32)
    div = jnp.sum(sim * offd) / (E_H * (E_H - 1))
    cov = jnp.sum(inc, axis=1, keepdims=True)
    covloss = jnp.mean((1.0 - jnp.clip(cov, 0.0, 1.0)) ** 2)
    return inc, div, covloss


def _head_body(ht_ref, hs_ref,
               j1w, j1b, j2w, j2b, d1w, d1b, d2w, d2b,
               g0w, g0b, g0e, g1w, g1b, g1e, g2w, g2b, g2e, g3w, g3b, g3e,
               ga1w, ga1b, ga2w, ga2b, de1w, de1b, de2w, de2b,
               c0w, c0b, c1w, c1b, p1w, p1b, p2w, p2b,
               preds_out, div_out, cov_out, sdl_out, beta_out):
    ht = ht_ref[0]
    hs = hs_ref[0]

    a = (jnp.dot(ht, j1w[:H], preferred_element_type=jnp.float32)
         + jnp.dot(hs, j1w[H:], preferred_element_type=jnp.float32) + j1b[...])
    joint = jnp.dot(_gelu(a), j2w[...], preferred_element_type=jnp.float32) + j2b[...]

    a = (jnp.dot(jnp.abs(ht - hs), d1w[:H], preferred_element_type=jnp.float32)
         + jnp.dot(ht * hs, d1w[H:], preferred_element_type=jnp.float32) + d1b[...])
    delta = jnp.dot(_gelu(a), d2w[...], preferred_element_type=jnp.float32) + d2b[...]

    shared, sdiv, scov = _generator(hs, g0w, g0b, g0e)
    inc1, dv1, cv1 = _generator(ht, g1w, g1b, g1e)
    inc2, dv2, cv2 = _generator(joint, g2w, g2b, g2e)
    inc3, dv3, cv3 = _generator(delta, g3w, g3b, g3e)

    sm = jnp.mean(hs, axis=0, keepdims=True)
    tm = jnp.mean(ht, axis=0, keepdims=True)
    jm = jnp.mean(joint, axis=0, keepdims=True)
    dm = jnp.mean(delta, axis=0, keepdims=True)
    dis = (jnp.abs(tm - sm) + jnp.abs(jm - sm) + jnp.abs(dm - sm)) / 3.0

    def mlp3(w_ref, b_ref):
        return (jnp.dot(sm, w_ref[:H], preferred_element_type=jnp.float32)
                + jnp.dot(tm, w_ref[H:2 * H], preferred_element_type=jnp.float32)
                + jnp.dot(dis, w_ref[2 * H:], preferred_element_type=jnp.float32)
                + b_ref[...])

    glog = (jnp.dot(_gelu(mlp3(ga1w, ga1b)), ga2w[...],
                    preferred_element_type=jnp.float32) + ga2b[...])
    lmin = jnp.min(glog, axis=1, keepdims=True)
    lmax = jnp.max(glog, axis=1, keepdims=True)
    ge = jnp.exp(glog - lmax) * (glog > lmin).astype(jnp.float32)
    probs = ge / jnp.sum(ge, axis=1, keepdims=True)

    bx = jnp.dot(_gelu(mlp3(de1w, de1b)), de2w[...],
                 preferred_element_type=jnp.float32) + de2b[...]
    beta = jax.nn.sigmoid(bx[0, 0])

    routed = probs[0, 0] * inc1 + probs[0, 1] * inc2 + probs[0, 2] * inc3
    inc = (1.0 - beta) * shared + beta * routed

    de = jnp.sum(inc, axis=0, keepdims=True) + 1e-6
    dv = jnp.sum(inc, axis=1, keepdims=True) + 1e-6
    inc_n = inc / de
    z = ht
    for cw, cb in ((c0w, c0b), (c1w, c1b)):
        m = jax.lax.dot_general(inc_n, z, (((0,), (0,)), ((), ())),
                                preferred_element_type=jnp.float32)
        out = jnp.dot(inc, m, preferred_element_type=jnp.float32) / dv
        z = _gelu(jnp.dot(out, cw[...], preferred_element_type=jnp.float32) + cb[...])

    pr = (jnp.dot(z, p1w[:H], preferred_element_type=jnp.float32)
          + jnp.dot(hs, p1w[H:], preferred_element_type=jnp.float32) + p1b[...])
    pr = jnp.dot(_gelu(pr), p2w[...], preferred_element_type=jnp.float32) + p2b[...]
    preds_out[...] = pr[None]

    n1 = jnp.sqrt(jnp.sum(inc1 * inc1)) + 1e-8
    n2 = jnp.sqrt(jnp.sum(inc2 * inc2)) + 1e-8
    n3 = jnp.sqrt(jnp.sum(inc3 * inc3)) + 1e-8
    s12 = jnp.sum(inc1 * inc2) / (n1 * n2)
    s13 = jnp.sum(inc1 * inc3) / (n1 * n3)
    s23 = jnp.sum(inc2 * inc3) / (n2 * n3)
    sdl = 2.0 * (s12 + s13 + s23) / 6.0

    div_out[...] = jnp.reshape(sdiv + (dv1 + dv2 + dv3) / 3.0, (1, 1, 1))
    cov_out[...] = jnp.reshape(scov + (cv1 + cv2 + cv3) / 3.0, (1, 1, 1))
    sdl_out[...] = jnp.reshape(sdl, (1, 1, 1))
    beta_out[...] = jnp.reshape(beta, (1, 1, 1))


def _run_head(ht3, hs3, p):
    const = lambda shape: pl.BlockSpec(shape, lambda b: (0,) * len(shape))
    gens = []
    for i in range(4):
        gens += [p['gen%d_proj' % i]['w'], p['gen%d_proj' % i]['b'][None, :],
                 p['gen%d_edges' % i].T]
    args = [
        ht3, hs3,
        p['joint1']['w'], p['joint1']['b'][None, :],
        p['joint2']['w'], p['joint2']['b'][None, :],
        p['delta1']['w'], p['delta1']['b'][None, :],
        p['delta2']['w'], p['delta2']['b'][None, :],
        *gens,
        p['gate1']['w'], p['gate1']['b'][None, :],
        p['gate2']['w'], p['gate2']['b'][None, :],
        p['dev1']['w'], p['dev1']['b'][None, :],
        p['dev2']['w'], p['dev2']['b'][None, :],
        p['conv0']['w'], p['conv0']['b'][None, :],
        p['conv1']['w'], p['conv1']['b'][None, :],
        p['pred1']['w'], p['pred1']['b'][None, :],
        p['pred2']['w'], p['pred2']['b'][None, :],
    ]
    in_specs = [
        pl.BlockSpec((1, N, H), lambda b: (b, 0, 0)),
        pl.BlockSpec((1, N, H), lambda b: (b, 0, 0)),
    ] + [const(a.shape) for a in args[2:]]
    outs = pl.pallas_call(
        _head_body,
        grid=(BSZ,),
        in_specs=in_specs,
        out_specs=[
            pl.BlockSpec((1, N, 1), lambda b: (b, 0, 0)),
            pl.BlockSpec((1, 1, 1), lambda b: (b, 0, 0)),
            pl.BlockSpec((1, 1, 1), lambda b: (b, 0, 0)),
            pl.BlockSpec((1, 1, 1), lambda b: (b, 0, 0)),
            pl.BlockSpec((1, 1, 1), lambda b: (b, 0, 0)),
        ],
        out_shape=[
            jax.ShapeDtypeStruct((BSZ, N, 1), jnp.float32),
            jax.ShapeDtypeStruct((BSZ, 1, 1), jnp.float32),
            jax.ShapeDtypeStruct((BSZ, 1, 1), jnp.float32),
            jax.ShapeDtypeStruct((BSZ, 1, 1), jnp.float32),
            jax.ShapeDtypeStruct((BSZ, 1, 1), jnp.float32),
        ],
        compiler_params=pltpu.CompilerParams(
            dimension_semantics=("arbitrary",)),
    )(*args)
    return outs


def kernel(forcing, static_attrs, params):
    f_t = jnp.transpose(forcing.reshape(BN, STEPS, FDIM), (1, 0, 2))
    s = static_attrs.reshape(BN, SDIM)
    hs, ht = _run_gru(f_t, s, params)
    preds3, div2, cov2, sdl2, beta2 = _run_head(
        ht.reshape(BSZ, N, H), hs.reshape(BSZ, N, H), params)
    preds = preds3.reshape(BSZ, N)
    inv = 1.0 / BSZ
    return (preds, jnp.sum(div2) * inv, jnp.sum(cov2) * inv,
            jnp.sum(sdl2) * inv, jnp.sum(beta2) * inv)


# speedup vs baseline: 1.1422x; 1.1422x over previous
import jax
import jax.numpy as jnp
import numpy as np
from jax.experimental import pallas as pl
from jax.experimental.pallas import tpu as pltpu

BSZ, N, STEPS, FDIM = 2, 512, 96, 16
SDIM, H, E_H, NSP, TOPK = 32, 256, 64, 3, 2
RATIO, MINE, MAXE, NLAYERS = 0.1, 8, 64, 2
KSZ = int(np.clip(int(N * RATIO), MINE, MAXE))
BN = BSZ * N

_SQRT_HALF = 0.7071067811865476


def _gelu(x):
    return 0.5 * x * (1.0 + jax.lax.erf(x * _SQRT_HALF))


UNROLL = 8


def _gru_body(f_ref, s_ref, ws1_ref, bs1_ref, ws2_ref, bs2_ref,
              wih_ref, whh_ref, brz_ref, bin_ref, bhn_ref,
              hs_out, ht_out, h1_scr, h2_scr):
    t = pl.program_id(0)
    HB = BN // 2

    @pl.when(t == 0)
    def _init():
        s = s_ref[...]
        h0 = _gelu(jnp.dot(s, ws1_ref[...], preferred_element_type=jnp.float32)
                   + bs1_ref[...])
        h0 = jnp.dot(h0, ws2_ref[...], preferred_element_type=jnp.float32) + bs2_ref[...]
        hs_out[...] = h0
        h1_scr[...] = h0[:HB]
        h2_scr[...] = h0[HB:]

    wih = wih_ref[...]
    whh = whh_ref[...]
    brz = brz_ref[...]
    bin_ = bin_ref[...]
    bhn = bhn_ref[...]
    hs = [h1_scr[...], h2_scr[...]]
    for j in range(UNROLL):
        for k, lo in enumerate((0, HB)):
            h = hs[k]
            xt = f_ref[j, pl.ds(lo, HB), :]
            gi = jnp.dot(xt, wih, preferred_element_type=jnp.float32)
            gh = jnp.dot(h, whh, preferred_element_type=jnp.float32)
            rz = jnp.tanh((gi[:, :2 * H] + gh[:, :2 * H] + brz) * 0.5)
            r = rz[:, :H] * 0.5 + 0.5
            z = rz[:, H:] * 0.5 + 0.5
            n = jnp.tanh(gi[:, 2 * H:] + bin_ + r * (gh[:, 2 * H:] + bhn))
            hs[k] = n + z * (h - n)
    h1_scr[...] = hs[0]
    h2_scr[...] = hs[1]

    @pl.when(t == STEPS // UNROLL - 1)
    def _fin():
        ht_out[pl.ds(0, HB), :] = hs[0]
        ht_out[pl.ds(HB, HB), :] = hs[1]


def _run_gru(f_t, s, p):
    const2 = lambda shape: pl.BlockSpec(shape, lambda t: (0, 0))
    grid = (STEPS // UNROLL,)
    out = pl.pallas_call(
        _gru_body,
        grid=grid,
        in_specs=[
            pl.BlockSpec((UNROLL, BN, FDIM), lambda t: (t, 0, 0)),
            const2((BN, SDIM)),
            const2((SDIM, H)), const2((1, H)),
            const2((H, H)), const2((1, H)),
            pl.BlockSpec((FDIM, 3 * H), lambda t: (0, 0)),
            pl.BlockSpec((H, 3 * H), lambda t: (0, 0)),
            const2((1, 2 * H)), const2((1, H)), const2((1, H)),
        ],
        out_specs=[
            pl.BlockSpec((BN, H), lambda t: (0, 0)),
            pl.BlockSpec((BN, H), lambda t: (0, 0)),
        ],
        out_shape=[
            jax.ShapeDtypeStruct((BN, H), jnp.float32),
            jax.ShapeDtypeStruct((BN, H), jnp.float32),
        ],
        scratch_shapes=[pltpu.VMEM((BN // 2, H), jnp.float32),
                        pltpu.VMEM((BN // 2, H), jnp.float32)],
        compiler_params=pltpu.CompilerParams(
            dimension_semantics=("arbitrary",)),
    )(
        f_t, s,
        p['static1']['w'], p['static1']['b'][None, :],
        p['static2']['w'], p['static2']['b'][None, :],
        p['gru_wih'], p['gru_whh'],
        (p['gru_bih'][:2 * H] + p['gru_bhh'][:2 * H])[None, :],
        p['gru_bih'][2 * H:][None, :], p['gru_bhh'][2 * H:][None, :],
    )
    return out


def _float_key(x):
    b = jax.lax.bitcast_convert_type(x, jnp.int32)
    mag = jnp.bitwise_and(b, jnp.int32(0x7FFFFFFF))
    return jnp.where(b < 0, -mag, b)


def _kth_threshold_keys(keys, k):
    cols = keys.shape[1]
    lo = jnp.full((1, cols), -2139095041, jnp.int32)
    hi = jnp.full((1, cols), 2139095041, jnp.int32)

    def body(_, lh):
        lo, hi = lh
        mid = jnp.bitwise_and(lo, hi) + (jnp.bitwise_xor(lo, hi) >> 1)
        cnt = jnp.sum((keys >= mid).astype(jnp.int32), axis=0, keepdims=True)
        take = cnt >= k
        return jnp.where(take, mid, lo), jnp.where(take, hi, mid)

    lo, hi = jax.lax.fori_loop(0, 33, body, (lo, hi))
    return lo


def _generator(view, pw_ref, pb_ref, pe_ref):
    q = _gelu(jnp.dot(view, pw_ref[...], preferred_element_type=jnp.float32)
              + pb_ref[...])
    logits = jnp.dot(q, pe_ref[...], preferred_element_type=jnp.float32)
    keys = _float_key(logits)
    thr = _kth_threshold_keys(keys, KSZ)
    sel = (keys >= thr).astype(jnp.float32)
    m = jnp.max(logits, axis=0, keepdims=True)
    e = jnp.exp(logits - m) * sel
    inc = e / jnp.sum(e, axis=0, keepdims=True)
    cnorm = jnp.sqrt(jnp.sum(inc * inc, axis=0, keepdims=True)) + 1e-8
    cols = inc / cnorm
    sim = jax.lax.dot_general(cols, cols, (((0,), (0,)), ((), ())),
                              preferred_element_type=jnp.float32)
    ri = jax.lax.broadcasted_iota(jnp.int32, (E_H, E_H), 0)
    ci = jax.lax.broadcasted_iota(jnp.int32, (E_H, E_H), 1)
    offd = (ri != ci).astype(jnp.float32)
    div = jnp.sum(sim * offd) / (E_H * (E_H - 1))
    cov = jnp.sum(inc, axis=1, keepdims=True)
    covloss = jnp.mean((1.0 - jnp.clip(cov, 0.0, 1.0)) ** 2)
    return inc, div, covloss


def _head_body(ht_ref, hs_ref,
               j1w, j1b, j2w, j2b, d1w, d1b, d2w, d2b,
               g0w, g0b, g0e, g1w, g1b, g1e, g2w, g2b, g2e, g3w, g3b, g3e,
               ga1w, ga1b, ga2w, ga2b, de1w, de1b, de2w, de2b,
               c0w, c0b, c1w, c1b, p1w, p1b, p2w, p2b,
               preds_out, div_out, cov_out, sdl_out, beta_out):
    ht = ht_ref[0]
    hs = hs_ref[0]

    a = (jnp.dot(ht, j1w[:H], preferred_element_type=jnp.float32)
         + jnp.dot(hs, j1w[H:], preferred_element_type=jnp.float32) + j1b[...])
    joint = jnp.dot(_gelu(a), j2w[...], preferred_element_type=jnp.float32) + j2b[...]

    a = (jnp.dot(jnp.abs(ht - hs), d1w[:H], preferred_element_type=jnp.float32)
         + jnp.dot(ht * hs, d1w[H:], preferred_element_type=jnp.float32) + d1b[...])
    delta = jnp.dot(_gelu(a), d2w[...], preferred_element_type=jnp.float32) + d2b[...]

    shared, sdiv, scov = _generator(hs, g0w, g0b, g0e)
    inc1, dv1, cv1 = _generator(ht, g1w, g1b, g1e)
    inc2, dv2, cv2 = _generator(joint, g2w, g2b, g2e)
    inc3, dv3, cv3 = _generator(delta, g3w, g3b, g3e)

    sm = jnp.mean(hs, axis=0, keepdims=True)
    tm = jnp.mean(ht, axis=0, keepdims=True)
    jm = jnp.mean(joint, axis=0, keepdims=True)
    dm = jnp.mean(delta, axis=0, keepdims=True)
    dis = (jnp.abs(tm - sm) + jnp.abs(jm - sm) + jnp.abs(dm - sm)) / 3.0

    def mlp3(w_ref, b_ref):
        return (jnp.dot(sm, w_ref[:H], preferred_element_type=jnp.float32)
                + jnp.dot(tm, w_ref[H:2 * H], preferred_element_type=jnp.float32)
                + jnp.dot(dis, w_ref[2 * H:], preferred_element_type=jnp.float32)
                + b_ref[...])

    glog = (jnp.dot(_gelu(mlp3(ga1w, ga1b)), ga2w[...],
                    preferred_element_type=jnp.float32) + ga2b[...])
    lmin = jnp.min(glog, axis=1, keepdims=True)
    lmax = jnp.max(glog, axis=1, keepdims=True)
    ge = jnp.exp(glog - lmax) * (glog > lmin).astype(jnp.float32)
    probs = ge / jnp.sum(ge, axis=1, keepdims=True)

    bx = jnp.dot(_gelu(mlp3(de1w, de1b)), de2w[...],
                 preferred_element_type=jnp.float32) + de2b[...]
    beta = jax.nn.sigmoid(bx[0, 0])

    routed = probs[0, 0] * inc1 + probs[0, 1] * inc2 + probs[0, 2] * inc3
    inc = (1.0 - beta) * shared + beta * routed

    de = jnp.sum(inc, axis=0, keepdims=True) + 1e-6
    dv = jnp.sum(inc, axis=1, keepdims=True) + 1e-6
    inc_n = inc / de
    z = ht
    for cw, cb in ((c0w, c0b), (c1w, c1b)):
        m = jax.lax.dot_general(inc_n, z, (((0,), (0,)), ((), ())),
                                preferred_element_type=jnp.float32)
        out = jnp.dot(inc, m, preferred_element_type=jnp.float32) / dv
        z = _gelu(jnp.dot(out, cw[...], preferred_element_type=jnp.float32) + cb[...])

    pr = (jnp.dot(z, p1w[:H], preferred_element_type=jnp.float32)
          + jnp.dot(hs, p1w[H:], preferred_element_type=jnp.float32) + p1b[...])
    pr = jnp.dot(_gelu(pr), p2w[...], preferred_element_type=jnp.float32) + p2b[...]
    preds_out[...] = pr[None]

    n1 = jnp.sqrt(jnp.sum(inc1 * inc1)) + 1e-8
    n2 = jnp.sqrt(jnp.sum(inc2 * inc2)) + 1e-8
    n3 = jnp.sqrt(jnp.sum(inc3 * inc3)) + 1e-8
    s12 = jnp.sum(inc1 * inc2) / (n1 * n2)
    s13 = jnp.sum(inc1 * inc3) / (n1 * n3)
    s23 = jnp.sum(inc2 * inc3) / (n2 * n3)
    sdl = 2.0 * (s12 + s13 + s23) / 6.0

    div_out[...] = jnp.reshape(sdiv + (dv1 + dv2 + dv3) / 3.0, (1, 1, 1))
    cov_out[...] = jnp.reshape(scov + (cv1 + cv2 + cv3) / 3.0, (1, 1, 1))
    sdl_out[...] = jnp.reshape(sdl, (1, 1, 1))
    beta_out[...] = jnp.reshape(beta, (1, 1, 1))


def _run_head(ht3, hs3, p):
    const = lambda shape: pl.BlockSpec(shape, lambda b: (0,) * len(shape))
    gens = []
    for i in range(4):
        gens += [p['gen%d_proj' % i]['w'], p['gen%d_proj' % i]['b'][None, :],
                 p['gen%d_edges' % i].T]
    args = [
        ht3, hs3,
        p['joint1']['w'], p['joint1']['b'][None, :],
        p['joint2']['w'], p['joint2']['b'][None, :],
        p['delta1']['w'], p['delta1']['b'][None, :],
        p['delta2']['w'], p['delta2']['b'][None, :],
        *gens,
        p['gate1']['w'], p['gate1']['b'][None, :],
        p['gate2']['w'], p['gate2']['b'][None, :],
        p['dev1']['w'], p['dev1']['b'][None, :],
        p['dev2']['w'], p['dev2']['b'][None, :],
        p['conv0']['w'], p['conv0']['b'][None, :],
        p['conv1']['w'], p['conv1']['b'][None, :],
        p['pred1']['w'], p['pred1']['b'][None, :],
        p['pred2']['w'], p['pred2']['b'][None, :],
    ]
    in_specs = [
        pl.BlockSpec((1, N, H), lambda b: (b, 0, 0)),
        pl.BlockSpec((1, N, H), lambda b: (b, 0, 0)),
    ] + [const(a.shape) for a in args[2:]]
    outs = pl.pallas_call(
        _head_body,
        grid=(BSZ,),
        in_specs=in_specs,
        out_specs=[
            pl.BlockSpec((1, N, 1), lambda b: (b, 0, 0)),
            pl.BlockSpec((1, 1, 1), lambda b: (b, 0, 0)),
            pl.BlockSpec((1, 1, 1), lambda b: (b, 0, 0)),
            pl.BlockSpec((1, 1, 1), lambda b: (b, 0, 0)),
            pl.BlockSpec((1, 1, 1), lambda b: (b, 0, 0)),
        ],
        out_shape=[
            jax.ShapeDtypeStruct((BSZ, N, 1), jnp.float32),
            jax.ShapeDtypeStruct((BSZ, 1, 1), jnp.float32),
            jax.ShapeDtypeStruct((BSZ, 1, 1), jnp.float32),
            jax.ShapeDtypeStruct((BSZ, 1, 1), jnp.float32),
            jax.ShapeDtypeStruct((BSZ, 1, 1), jnp.float32),
        ],
        compiler_params=pltpu.CompilerParams(
            dimension_semantics=("arbitrary",)),
    )(*args)
    return outs


def kernel(forcing, static_attrs, params):
    f_t = jnp.transpose(forcing.reshape(BN, STEPS, FDIM), (1, 0, 2))
    s = static_attrs.reshape(BN, SDIM)
    hs, ht = _run_gru(f_t, s, params)
    preds3, div2, cov2, sdl2, beta2 = _run_head(
        ht.reshape(BSZ, N, H), hs.reshape(BSZ, N, H), params)
    preds = preds3.reshape(BSZ, N)
    inv = 1.0 / BSZ
    return (preds, jnp.sum(div2) * inv, jnp.sum(cov2) * inv,
            jnp.sum(sdl2) * inv, jnp.sum(beta2) * inv)


# speedup vs baseline: 1.1531x; 1.0095x over previous
import jax
import jax.numpy as jnp
import numpy as np
from jax.experimental import pallas as pl
from jax.experimental.pallas import tpu as pltpu

BSZ, N, STEPS, FDIM = 2, 512, 96, 16
SDIM, H, E_H, NSP, TOPK = 32, 256, 64, 3, 2
RATIO, MINE, MAXE, NLAYERS = 0.1, 8, 64, 2
KSZ = int(np.clip(int(N * RATIO), MINE, MAXE))
BN = BSZ * N

_SQRT_HALF = 0.7071067811865476


def _gelu(x):
    return 0.5 * x * (1.0 + jax.lax.erf(x * _SQRT_HALF))


UNROLL = 8


def _gru_body(f_ref, s_ref, ws1_ref, bs1_ref, ws2_ref, bs2_ref,
              wih_ref, whh_ref, brz_ref, bin_ref, bhn_ref,
              hs_out, ht_out, h1_scr, h2_scr):
    t = pl.program_id(0)
    HB = BN // 2

    @pl.when(t == 0)
    def _init():
        s = s_ref[...]
        h0 = _gelu(jnp.dot(s, ws1_ref[...], preferred_element_type=jnp.float32)
                   + bs1_ref[...])
        h0 = jnp.dot(h0, ws2_ref[...], preferred_element_type=jnp.float32) + bs2_ref[...]
        hs_out[...] = h0
        h1_scr[...] = h0[:HB]
        h2_scr[...] = h0[HB:]

    wih = wih_ref[...]
    whh = whh_ref[...]
    brz = brz_ref[...]
    bin_ = bin_ref[...]
    bhn = bhn_ref[...]
    hs = [h1_scr[...], h2_scr[...]]
    for j in range(UNROLL):
        for k, lo in enumerate((0, HB)):
            h = hs[k]
            xt = f_ref[j, pl.ds(lo, HB), :]
            gi = jnp.dot(xt, wih, preferred_element_type=jnp.float32)
            gh = jnp.dot(h, whh, preferred_element_type=jnp.float32)
            rz = jax.nn.sigmoid(gi[:, :2 * H] + gh[:, :2 * H] + brz)
            r = rz[:, :H]
            z = rz[:, H:]
            n = jnp.tanh(gi[:, 2 * H:] + bin_ + r * (gh[:, 2 * H:] + bhn))
            hs[k] = n + z * (h - n)
    h1_scr[...] = hs[0]
    h2_scr[...] = hs[1]

    @pl.when(t == STEPS // UNROLL - 1)
    def _fin():
        ht_out[pl.ds(0, HB), :] = hs[0]
        ht_out[pl.ds(HB, HB), :] = hs[1]


def _run_gru(f_t, s, p):
    const2 = lambda shape: pl.BlockSpec(shape, lambda t: (0, 0))
    grid = (STEPS // UNROLL,)
    out = pl.pallas_call(
        _gru_body,
        grid=grid,
        in_specs=[
            pl.BlockSpec((UNROLL, BN, FDIM), lambda t: (t, 0, 0)),
            const2((BN, SDIM)),
            const2((SDIM, H)), const2((1, H)),
            const2((H, H)), const2((1, H)),
            pl.BlockSpec((FDIM, 3 * H), lambda t: (0, 0)),
            pl.BlockSpec((H, 3 * H), lambda t: (0, 0)),
            const2((1, 2 * H)), const2((1, H)), const2((1, H)),
        ],
        out_specs=[
            pl.BlockSpec((BN, H), lambda t: (0, 0)),
            pl.BlockSpec((BN, H), lambda t: (0, 0)),
        ],
        out_shape=[
            jax.ShapeDtypeStruct((BN, H), jnp.float32),
            jax.ShapeDtypeStruct((BN, H), jnp.float32),
        ],
        scratch_shapes=[pltpu.VMEM((BN // 2, H), jnp.float32),
                        pltpu.VMEM((BN // 2, H), jnp.float32)],
        compiler_params=pltpu.CompilerParams(
            dimension_semantics=("arbitrary",)),
    )(
        f_t, s,
        p['static1']['w'], p['static1']['b'][None, :],
        p['static2']['w'], p['static2']['b'][None, :],
        p['gru_wih'], p['gru_whh'],
        (p['gru_bih'][:2 * H] + p['gru_bhh'][:2 * H])[None, :],
        p['gru_bih'][2 * H:][None, :], p['gru_bhh'][2 * H:][None, :],
    )
    return out


def _float_key(x):
    b = jax.lax.bitcast_convert_type(x, jnp.int32)
    mag = jnp.bitwise_and(b, jnp.int32(0x7FFFFFFF))
    return jnp.where(b < 0, -mag, b)


def _kth_threshold_keys(keys, k):
    cols = keys.shape[1]
    lo = jnp.full((1, cols), -2139095041, jnp.int32)
    hi = jnp.full((1, cols), 2139095041, jnp.int32)

    def body(_, lh):
        lo, hi = lh
        mid = jnp.bitwise_and(lo, hi) + (jnp.bitwise_xor(lo, hi) >> 1)
        cnt = jnp.sum((keys >= mid).astype(jnp.int32), axis=0, keepdims=True)
        take = cnt >= k
        return jnp.where(take, mid, lo), jnp.where(take, hi, mid)

    lo, hi = jax.lax.fori_loop(0, 33, body, (lo, hi))
    return lo


def _generator(view, pw_ref, pb_ref, pe_ref):
    q = _gelu(jnp.dot(view, pw_ref[...], preferred_element_type=jnp.float32)
              + pb_ref[...])
    logits = jnp.dot(q, pe_ref[...], preferred_element_type=jnp.float32)
    keys = _float_key(logits)
    thr = _kth_threshold_keys(keys, KSZ)
    sel = (keys >= thr).astype(jnp.float32)
    m = jnp.max(logits, axis=0, keepdims=True)
    e = jnp.exp(logits - m) * sel
    inc = e / jnp.sum(e, axis=0, keepdims=True)
    cnorm = jnp.sqrt(jnp.sum(inc * inc, axis=0, keepdims=True)) + 1e-8
    cols = inc / cnorm
    sim = jax.lax.dot_general(cols, cols, (((0,), (0,)), ((), ())),
                              preferred_element_type=jnp.float32)
    ri = jax.lax.broadcasted_iota(jnp.int32, (E_H, E_H), 0)
    ci = jax.lax.broadcasted_iota(jnp.int32, (E_H, E_H), 1)
    offd = (ri != ci).astype(jnp.float32)
    div = jnp.sum(sim * offd) / (E_H * (E_H - 1))
    cov = jnp.sum(inc, axis=1, keepdims=True)
    covloss = jnp.mean((1.0 - jnp.clip(cov, 0.0, 1.0)) ** 2)
    return inc, div, covloss


def _head_body(ht_ref, hs_ref,
               j1w, j1b, j2w, j2b, d1w, d1b, d2w, d2b,
               g0w, g0b, g0e, g1w, g1b, g1e, g2w, g2b, g2e, g3w, g3b, g3e,
               ga1w, ga1b, ga2w, ga2b, de1w, de1b, de2w, de2b,
               c0w, c0b, c1w, c1b, p1w, p1b, p2w, p2b,
               preds_out, div_out, cov_out, sdl_out, beta_out):
    ht = ht_ref[0]
    hs = hs_ref[0]

    a = (jnp.dot(ht, j1w[:H], preferred_element_type=jnp.float32)
         + jnp.dot(hs, j1w[H:], preferred_element_type=jnp.float32) + j1b[...])
    joint = jnp.dot(_gelu(a), j2w[...], preferred_element_type=jnp.float32) + j2b[...]

    a = (jnp.dot(jnp.abs(ht - hs), d1w[:H], preferred_element_type=jnp.float32)
         + jnp.dot(ht * hs, d1w[H:], preferred_element_type=jnp.float32) + d1b[...])
    delta = jnp.dot(_gelu(a), d2w[...], preferred_element_type=jnp.float32) + d2b[...]

    shared, sdiv, scov = _generator(hs, g0w, g0b, g0e)
    inc1, dv1, cv1 = _generator(ht, g1w, g1b, g1e)
    inc2, dv2, cv2 = _generator(joint, g2w, g2b, g2e)
    inc3, dv3, cv3 = _generator(delta, g3w, g3b, g3e)

    sm = jnp.mean(hs, axis=0, keepdims=True)
    tm = jnp.mean(ht, axis=0, keepdims=True)
    jm = jnp.mean(joint, axis=0, keepdims=True)
    dm = jnp.mean(delta, axis=0, keepdims=True)
    dis = (jnp.abs(tm - sm) + jnp.abs(jm - sm) + jnp.abs(dm - sm)) / 3.0

    def mlp3(w_ref, b_ref):
        return (jnp.dot(sm, w_ref[:H], preferred_element_type=jnp.float32)
                + jnp.dot(tm, w_ref[H:2 * H], preferred_element_type=jnp.float32)
                + jnp.dot(dis, w_ref[2 * H:], preferred_element_type=jnp.float32)
                + b_ref[...])

    glog = (jnp.dot(_gelu(mlp3(ga1w, ga1b)), ga2w[...],
                    preferred_element_type=jnp.float32) + ga2b[...])
    lmin = jnp.min(glog, axis=1, keepdims=True)
    lmax = jnp.max(glog, axis=1, keepdims=True)
    ge = jnp.exp(glog - lmax) * (glog > lmin).astype(jnp.float32)
    probs = ge / jnp.sum(ge, axis=1, keepdims=True)

    bx = jnp.dot(_gelu(mlp3(de1w, de1b)), de2w[...],
                 preferred_element_type=jnp.float32) + de2b[...]
    beta = jax.nn.sigmoid(bx[0, 0])

    routed = probs[0, 0] * inc1 + probs[0, 1] * inc2 + probs[0, 2] * inc3
    inc = (1.0 - beta) * shared + beta * routed

    de = jnp.sum(inc, axis=0, keepdims=True) + 1e-6
    dv = jnp.sum(inc, axis=1, keepdims=True) + 1e-6
    inc_n = inc / de
    z = ht
    for cw, cb in ((c0w, c0b), (c1w, c1b)):
        m = jax.lax.dot_general(inc_n, z, (((0,), (0,)), ((), ())),
                                preferred_element_type=jnp.float32)
        out = jnp.dot(inc, m, preferred_element_type=jnp.float32) / dv
        z = _gelu(jnp.dot(out, cw[...], preferred_element_type=jnp.float32) + cb[...])

    pr = (jnp.dot(z, p1w[:H], preferred_element_type=jnp.float32)
          + jnp.dot(hs, p1w[H:], preferred_element_type=jnp.float32) + p1b[...])
    pr = jnp.dot(_gelu(pr), p2w[...], preferred_element_type=jnp.float32) + p2b[...]
    preds_out[...] = pr[None]

    n1 = jnp.sqrt(jnp.sum(inc1 * inc1)) + 1e-8
    n2 = jnp.sqrt(jnp.sum(inc2 * inc2)) + 1e-8
    n3 = jnp.sqrt(jnp.sum(inc3 * inc3)) + 1e-8
    s12 = jnp.sum(inc1 * inc2) / (n1 * n2)
    s13 = jnp.sum(inc1 * inc3) / (n1 * n3)
    s23 = jnp.sum(inc2 * inc3) / (n2 * n3)
    sdl = 2.0 * (s12 + s13 + s23) / 6.0

    div_out[...] = jnp.reshape(sdiv + (dv1 + dv2 + dv3) / 3.0, (1, 1, 1))
    cov_out[...] = jnp.reshape(scov + (cv1 + cv2 + cv3) / 3.0, (1, 1, 1))
    sdl_out[...] = jnp.reshape(sdl, (1, 1, 1))
    beta_out[...] = jnp.reshape(beta, (1, 1, 1))


def _run_head(ht3, hs3, p):
    const = lambda shape: pl.BlockSpec(shape, lambda b: (0,) * len(shape))
    gens = []
    for i in range(4):
        gens += [p['gen%d_proj' % i]['w'], p['gen%d_proj' % i]['b'][None, :],
                 p['gen%d_edges' % i].T]
    args = [
        ht3, hs3,
        p['joint1']['w'], p['joint1']['b'][None, :],
        p['joint2']['w'], p['joint2']['b'][None, :],
        p['delta1']['w'], p['delta1']['b'][None, :],
        p['delta2']['w'], p['delta2']['b'][None, :],
        *gens,
        p['gate1']['w'], p['gate1']['b'][None, :],
        p['gate2']['w'], p['gate2']['b'][None, :],
        p['dev1']['w'], p['dev1']['b'][None, :],
        p['dev2']['w'], p['dev2']['b'][None, :],
        p['conv0']['w'], p['conv0']['b'][None, :],
        p['conv1']['w'], p['conv1']['b'][None, :],
        p['pred1']['w'], p['pred1']['b'][None, :],
        p['pred2']['w'], p['pred2']['b'][None, :],
    ]
    in_specs = [
        pl.BlockSpec((1, N, H), lambda b: (b, 0, 0)),
        pl.BlockSpec((1, N, H), lambda b: (b, 0, 0)),
    ] + [const(a.shape) for a in args[2:]]
    outs = pl.pallas_call(
        _head_body,
        grid=(BSZ,),
        in_specs=in_specs,
        out_specs=[
            pl.BlockSpec((1, N, 1), lambda b: (b, 0, 0)),
            pl.BlockSpec((1, 1, 1), lambda b: (b, 0, 0)),
            pl.BlockSpec((1, 1, 1), lambda b: (b, 0, 0)),
            pl.BlockSpec((1, 1, 1), lambda b: (b, 0, 0)),
            pl.BlockSpec((1, 1, 1), lambda b: (b, 0, 0)),
        ],
        out_shape=[
            jax.ShapeDtypeStruct((BSZ, N, 1), jnp.float32),
            jax.ShapeDtypeStruct((BSZ, 1, 1), jnp.float32),
            jax.ShapeDtypeStruct((BSZ, 1, 1), jnp.float32),
            jax.ShapeDtypeStruct((BSZ, 1, 1), jnp.float32),
            jax.ShapeDtypeStruct((BSZ, 1, 1), jnp.float32),
        ],
        compiler_params=pltpu.CompilerParams(
            dimension_semantics=("arbitrary",)),
    )(*args)
    return outs


def kernel(forcing, static_attrs, params):
    f_t = jnp.transpose(forcing.reshape(BN, STEPS, FDIM), (1, 0, 2))
    s = static_attrs.reshape(BN, SDIM)
    hs, ht = _run_gru(f_t, s, params)
    preds3, div2, cov2, sdl2, beta2 = _run_head(
        ht.reshape(BSZ, N, H), hs.reshape(BSZ, N, H), params)
    preds = preds3.reshape(BSZ, N)
    inv = 1.0 / BSZ
    return (preds, jnp.sum(div2) * inv, jnp.sum(cov2) * inv,
            jnp.sum(sdl2) * inv, jnp.sum(beta2) * inv)
